# baseline (speedup 1.0000x reference)
import jax
import jax.numpy as jnp
from jax import lax
from jax.experimental import pallas as pl
from jax.experimental.pallas import tpu as pltpu
from jax.experimental.pallas import tpu_sc as plsc

NC = 2
NS = 16
NW = NC * NS
D = 64
CHUNK = 128
SUB = 4
STAGE = CHUNK * SUB
K = 5


TK = 8192
HF = TK // 2


def _pack_body(x_ref, i_ref, y_ref):
    x = x_ref[...]
    x2 = jnp.concatenate([x[:, :HF], x[:, HF:]], axis=0)
    y_ref[...] = lax.dot_general(
        x2, i_ref[...], (((0,), (0,)), ((), ())),
        preferred_element_type=jnp.float32)


def _pack_table(tableT, ident):
    _, v = tableT.shape
    nblk = pl.cdiv(v, TK)
    return pl.pallas_call(
        _pack_body,
        grid=(nblk,),
        in_specs=[
            pl.BlockSpec((D, TK), lambda i: (0, i)),
            pl.BlockSpec((2 * D, 2 * D), lambda i: (0, 0)),
        ],
        out_specs=pl.BlockSpec((HF, 2 * D), lambda i: (i, 0)),
        out_shape=jax.ShapeDtypeStruct((nblk * HF, 2 * D), jnp.float32),
        compiler_params=pltpu.CompilerParams(fuse_transposed_lhs_in_matmul=True),
        name="table_pack_rowmajor",
    )(tableT, ident)


def _gather_body(table_hbm, idx_hbm, out_hbm,
                 idx_v, rows_a, rows_b, sem_a, sem_b, sem_sa, sem_sb):
    wid = lax.axis_index("s") * NC + lax.axis_index("c")
    chunks_per_w = idx_v.shape[0]
    n_per_w = chunks_per_w * CHUNK
    n_stages = chunks_per_w // SUB
    base = wid * n_per_w

    pltpu.sync_copy(idx_hbm.at[pl.ds(wid * chunks_per_w, chunks_per_w)], idx_v)

    def fire(stage, rows_v, sem):
        for j in range(SUB):
            pltpu.async_copy(
                table_hbm.at[idx_v.at[stage * SUB + j]],
                rows_v.at[pl.ds(j * CHUNK, CHUNK)],
                sem,
            )

    def wait_gathers(stage, rows_v, sem):
        for j in range(SUB):
            pltpu.make_async_copy(
                table_hbm.at[idx_v.at[stage * SUB + j]],
                rows_v.at[pl.ds(j * CHUNK, CHUNK)],
                sem,
            ).wait()

    def store(stage, rows_v, sem):
        pltpu.async_copy(
            rows_v, out_hbm.at[pl.ds(base + stage * STAGE, STAGE)], sem
        )

    def wait_store(stage, rows_v, sem):
        pltpu.make_async_copy(
            rows_v, out_hbm.at[pl.ds(base + stage * STAGE, STAGE)], sem
        ).wait()

    fire(0, rows_a, sem_a)
    fire(1, rows_b, sem_b)
    wait_gathers(0, rows_a, sem_a)
    store(0, rows_a, sem_sa)

    def one_stage(s, cur, cur_gsem, cur_ssem, other, other_gsem, other_ssem):
        wait_store(s - 1, other, other_ssem)

        @pl.when(s + 1 < n_stages)
        def _():
            fire(s + 1, other, other_gsem)

        wait_gathers(s, cur, cur_gsem)
        store(s, cur, cur_ssem)

    def pair(k, carry):
        s = 2 * k + 1
        one_stage(s, rows_b, sem_b, sem_sb, rows_a, sem_a, sem_sa)
        one_stage(s + 1, rows_a, sem_a, sem_sa, rows_b, sem_b, sem_sb)
        return carry

    lax.fori_loop(0, (n_stages - 2) // 2, pair, 0, unroll=False)
    s_last = n_stages - 1
    one_stage(s_last, rows_b, sem_b, sem_sb, rows_a, sem_a, sem_sa)
    wait_store(s_last, rows_b, sem_sb)


def _sc_gather(yv, idx2d):
    n_chunks = idx2d.shape[0]
    n = n_chunks * CHUNK
    chunks_per_w = n_chunks // NW
    mesh = plsc.VectorSubcoreMesh(
        core_axis_name="c", subcore_axis_name="s", num_cores=NC, num_subcores=NS
    )
    return pl.kernel(
        _gather_body,
        out_type=jax.ShapeDtypeStruct((n, D), jnp.float32),
        mesh=mesh,
        scratch_types=[
            pltpu.VMEM((chunks_per_w, CHUNK), jnp.int32),
            pltpu.VMEM((STAGE, D), jnp.float32),
            pltpu.VMEM((STAGE, D), jnp.float32),
            pltpu.SemaphoreType.DMA,
            pltpu.SemaphoreType.DMA,
            pltpu.SemaphoreType.DMA,
            pltpu.SemaphoreType.DMA,
        ],
        compiler_params=pltpu.CompilerParams(use_tc_tiling_on_sc=False),
        name="sc_embedding_gather",
    )(yv, idx2d)


def _project_body(x_ref, w_ref, b_ref, *rest):
    o_ref = rest[-1]
    z2t = lax.dot_general(
        w_ref[...], x_ref[...], (((0,), (1,)), ((), ())),
        preferred_element_type=jnp.float32)
    y = jnp.tanh(z2t + b_ref[...])
    o_ref[0] = y[:D]
    o_ref[1] = y[D:]


def _project_chunk(g2c, Wd, b2, prev, k, pairs, B, L):
    in_specs = [
        pl.BlockSpec((B, 2 * D), lambda i: (i, 0)),
        pl.BlockSpec((2 * D, 2 * D), lambda i: (0, 0)),
        pl.BlockSpec((2 * D, 1), lambda i: (0, 0)),
    ]
    operands = [g2c, Wd, b2]
    aliases = {}
    if prev is not None:
        in_specs.append(pl.BlockSpec(memory_space=pl.ANY))
        operands.append(prev)
        aliases = {3: 0}
    return pl.pallas_call(
        _project_body,
        grid=(pairs,),
        in_specs=in_specs,
        out_specs=pl.BlockSpec((2, D, B), lambda i: (k * pairs + i, 0, 0)),
        out_shape=jax.ShapeDtypeStruct((L, D, B), jnp.float32),
        input_output_aliases=aliases,
        name="project_tanh_to_lanes",
    )(*operands)


def kernel(inputs, table, W, b):
    B, L = inputs.shape
    tableT = table.T
    ident = jnp.eye(2 * D, dtype=jnp.float32)
    y2 = _pack_table(tableT, ident)
    yv = y2.reshape(-1, D)

    idxp = inputs.T.astype(jnp.int32)
    idx_pair = jnp.transpose(
        idxp.reshape(L // 2, 2, B), (0, 2, 1)).reshape(-1)
    q = ((idx_pair >> 13) << 12) + (idx_pair & (HF - 1))
    idx_r = 2 * q + ((idx_pair >> 12) & 1)

    Wd = jnp.zeros((2 * D, 2 * D), jnp.float32)
    Wd = Wd.at[:D, :D].set(W).at[D:, D:].set(W)
    b2 = jnp.concatenate([b, b]).reshape(2 * D, 1)

    pairs = L // 2 // K
    rows = pairs * 2 * B
    out = None
    for k in range(K):
        idx2d = lax.slice(idx_r, (k * rows,), ((k + 1) * rows,)).reshape(
            -1, CHUNK)
        g = _sc_gather(yv, idx2d)
        g2 = g.reshape(rows // 2, 2 * D)
        out = _project_chunk(g2, Wd, b2, out, k, pairs, B, L)

    return jnp.transpose(out, (2, 0, 1))


# speedup vs baseline: 1.0411x; 1.0411x over previous
import jax
import jax.numpy as jnp
from jax import lax
from jax.experimental import pallas as pl
from jax.experimental.pallas import tpu as pltpu
from jax.experimental.pallas import tpu_sc as plsc

NC = 2
NS = 16
NW = NC * NS
D = 64
CHUNK = 128
SUB = 4
STAGE = CHUNK * SUB
K = 1


TK = 8192
HF = TK // 2


def _pack_body(x_ref, i_ref, y_ref):
    x = x_ref[...]
    x2 = jnp.concatenate([x[:, :HF], x[:, HF:]], axis=0)
    y_ref[...] = lax.dot_general(
        x2, i_ref[...], (((0,), (0,)), ((), ())),
        preferred_element_type=jnp.float32)


def _pack_table(tableT, ident):
    _, v = tableT.shape
    nblk = pl.cdiv(v, TK)
    return pl.pallas_call(
        _pack_body,
        grid=(nblk,),
        in_specs=[
            pl.BlockSpec((D, TK), lambda i: (0, i)),
            pl.BlockSpec((2 * D, 2 * D), lambda i: (0, 0)),
        ],
        out_specs=pl.BlockSpec((HF, 2 * D), lambda i: (i, 0)),
        out_shape=jax.ShapeDtypeStruct((nblk * HF, 2 * D), jnp.float32),
        compiler_params=pltpu.CompilerParams(fuse_transposed_lhs_in_matmul=True),
        name="table_pack_rowmajor",
    )(tableT, ident)


def _gather_body(table_hbm, idx_hbm, out_hbm,
                 idx_v, rows_a, rows_b, sem_a, sem_b, sem_sa, sem_sb):
    wid = lax.axis_index("s") * NC + lax.axis_index("c")
    chunks_per_w = idx_v.shape[0]
    n_per_w = chunks_per_w * CHUNK
    n_stages = chunks_per_w // SUB
    base = wid * n_per_w

    pltpu.sync_copy(idx_hbm.at[pl.ds(wid * chunks_per_w, chunks_per_w)], idx_v)

    def fire(stage, rows_v, sem):
        for j in range(SUB):
            pltpu.async_copy(
                table_hbm.at[idx_v.at[stage * SUB + j]],
                rows_v.at[pl.ds(j * CHUNK, CHUNK)],
                sem,
            )

    def wait_gathers(stage, rows_v, sem):
        for j in range(SUB):
            pltpu.make_async_copy(
                table_hbm.at[idx_v.at[stage * SUB + j]],
                rows_v.at[pl.ds(j * CHUNK, CHUNK)],
                sem,
            ).wait()

    def store(stage, rows_v, sem):
        pltpu.async_copy(
            rows_v, out_hbm.at[pl.ds(base + stage * STAGE, STAGE)], sem
        )

    def wait_store(stage, rows_v, sem):
        pltpu.make_async_copy(
            rows_v, out_hbm.at[pl.ds(base + stage * STAGE, STAGE)], sem
        ).wait()

    fire(0, rows_a, sem_a)
    fire(1, rows_b, sem_b)
    wait_gathers(0, rows_a, sem_a)
    store(0, rows_a, sem_sa)

    def one_stage(s, cur, cur_gsem, cur_ssem, other, other_gsem, other_ssem):
        wait_store(s - 1, other, other_ssem)

        @pl.when(s + 1 < n_stages)
        def _():
            fire(s + 1, other, other_gsem)

        wait_gathers(s, cur, cur_gsem)
        store(s, cur, cur_ssem)

    def pair(k, carry):
        s = 2 * k + 1
        one_stage(s, rows_b, sem_b, sem_sb, rows_a, sem_a, sem_sa)
        one_stage(s + 1, rows_a, sem_a, sem_sa, rows_b, sem_b, sem_sb)
        return carry

    lax.fori_loop(0, (n_stages - 2) // 2, pair, 0, unroll=False)
    s_last = n_stages - 1
    one_stage(s_last, rows_b, sem_b, sem_sb, rows_a, sem_a, sem_sa)
    wait_store(s_last, rows_b, sem_sb)


def _sc_gather(yv, idx2d):
    n_chunks = idx2d.shape[0]
    n = n_chunks * CHUNK
    chunks_per_w = n_chunks // NW
    mesh = plsc.VectorSubcoreMesh(
        core_axis_name="c", subcore_axis_name="s", num_cores=NC, num_subcores=NS
    )
    return pl.kernel(
        _gather_body,
        out_type=jax.ShapeDtypeStruct((n, D), jnp.float32),
        mesh=mesh,
        scratch_types=[
            pltpu.VMEM((chunks_per_w, CHUNK), jnp.int32),
            pltpu.VMEM((STAGE, D), jnp.float32),
            pltpu.VMEM((STAGE, D), jnp.float32),
            pltpu.SemaphoreType.DMA,
            pltpu.SemaphoreType.DMA,
            pltpu.SemaphoreType.DMA,
            pltpu.SemaphoreType.DMA,
        ],
        compiler_params=pltpu.CompilerParams(use_tc_tiling_on_sc=False),
        name="sc_embedding_gather",
    )(yv, idx2d)


def _project_body(x_ref, w_ref, b_ref, *rest):
    o_ref = rest[-1]
    z2t = lax.dot_general(
        w_ref[...], x_ref[...], (((0,), (1,)), ((), ())),
        preferred_element_type=jnp.float32)
    y = jnp.tanh(z2t + b_ref[...])
    o_ref[0] = y[:D]
    o_ref[1] = y[D:]


def _project_chunk(g2c, Wd, b2, prev, k, pairs, B, L):
    in_specs = [
        pl.BlockSpec((B, 2 * D), lambda i: (i, 0)),
        pl.BlockSpec((2 * D, 2 * D), lambda i: (0, 0)),
        pl.BlockSpec((2 * D, 1), lambda i: (0, 0)),
    ]
    operands = [g2c, Wd, b2]
    aliases = {}
    if prev is not None:
        in_specs.append(pl.BlockSpec(memory_space=pl.ANY))
        operands.append(prev)
        aliases = {3: 0}
    return pl.pallas_call(
        _project_body,
        grid=(pairs,),
        in_specs=in_specs,
        out_specs=pl.BlockSpec((2, D, B), lambda i: (k * pairs + i, 0, 0)),
        out_shape=jax.ShapeDtypeStruct((L, D, B), jnp.float32),
        input_output_aliases=aliases,
        name="project_tanh_to_lanes",
    )(*operands)


def kernel(inputs, table, W, b):
    B, L = inputs.shape
    tableT = table.T
    ident = jnp.eye(2 * D, dtype=jnp.float32)
    y2 = _pack_table(tableT, ident)
    yv = y2.reshape(-1, D)

    idxp = inputs.T.astype(jnp.int32)
    idx_pair = jnp.transpose(
        idxp.reshape(L // 2, 2, B), (0, 2, 1)).reshape(-1)
    q = ((idx_pair >> 13) << 12) + (idx_pair & (HF - 1))
    idx_r = 2 * q + ((idx_pair >> 12) & 1)

    Wd = jnp.zeros((2 * D, 2 * D), jnp.float32)
    Wd = Wd.at[:D, :D].set(W).at[D:, D:].set(W)
    b2 = jnp.concatenate([b, b]).reshape(2 * D, 1)

    pairs = L // 2 // K
    rows = pairs * 2 * B
    out = None
    for k in range(K):
        idx2d = lax.slice(idx_r, (k * rows,), ((k + 1) * rows,)).reshape(
            -1, CHUNK)
        g = _sc_gather(yv, idx2d)
        g2 = g.reshape(rows // 2, 2 * D)
        out = _project_chunk(g2, Wd, b2, out, k, pairs, B, L)

    return jnp.transpose(out, (2, 0, 1))


# speedup vs baseline: 1.0796x; 1.0370x over previous
import jax
import jax.numpy as jnp
from jax import lax
from jax.experimental import pallas as pl
from jax.experimental.pallas import tpu as pltpu
from jax.experimental.pallas import tpu_sc as plsc

NC = 2
NS = 16
NW = NC * NS
D = 64
CHUNK = 128
SUB = 4
STAGE = CHUNK * SUB
K = 1


TK = 16384
HF = TK // 2
SH_B = TK.bit_length() - 1
SH_H = HF.bit_length() - 1


def _pack_body(x_ref, i_ref, y_ref):
    x = x_ref[...]
    x2 = jnp.concatenate([x[:, :HF], x[:, HF:]], axis=0)
    y_ref[...] = lax.dot_general(
        x2, i_ref[...], (((0,), (0,)), ((), ())),
        preferred_element_type=jnp.float32)


def _pack_table(tableT, ident):
    _, v = tableT.shape
    nblk = pl.cdiv(v, TK)
    return pl.pallas_call(
        _pack_body,
        grid=(nblk,),
        in_specs=[
            pl.BlockSpec((D, TK), lambda i: (0, i)),
            pl.BlockSpec((2 * D, 2 * D), lambda i: (0, 0)),
        ],
        out_specs=pl.BlockSpec((HF, 2 * D), lambda i: (i, 0)),
        out_shape=jax.ShapeDtypeStruct((nblk * HF, 2 * D), jnp.float32),
        compiler_params=pltpu.CompilerParams(fuse_transposed_lhs_in_matmul=True),
        name="table_pack_rowmajor",
    )(tableT, ident)


def _gather_body(table_hbm, idx_hbm, out_hbm,
                 idx_v, rows_a, rows_b, sem_a, sem_b, sem_sa, sem_sb):
    wid = lax.axis_index("s") * NC + lax.axis_index("c")
    chunks_per_w = idx_v.shape[0]
    n_per_w = chunks_per_w * CHUNK
    n_stages = chunks_per_w // SUB
    base = wid * n_per_w

    pltpu.sync_copy(idx_hbm.at[pl.ds(wid * chunks_per_w, chunks_per_w)], idx_v)

    def fire(stage, rows_v, sem):
        for j in range(SUB):
            pltpu.async_copy(
                table_hbm.at[idx_v.at[stage * SUB + j]],
                rows_v.at[pl.ds(j * CHUNK, CHUNK)],
                sem,
            )

    def wait_gathers(stage, rows_v, sem):
        for j in range(SUB):
            pltpu.make_async_copy(
                table_hbm.at[idx_v.at[stage * SUB + j]],
                rows_v.at[pl.ds(j * CHUNK, CHUNK)],
                sem,
            ).wait()

    def store(stage, rows_v, sem):
        pltpu.async_copy(
            rows_v, out_hbm.at[pl.ds(base + stage * STAGE, STAGE)], sem
        )

    def wait_store(stage, rows_v, sem):
        pltpu.make_async_copy(
            rows_v, out_hbm.at[pl.ds(base + stage * STAGE, STAGE)], sem
        ).wait()

    fire(0, rows_a, sem_a)
    fire(1, rows_b, sem_b)
    wait_gathers(0, rows_a, sem_a)
    store(0, rows_a, sem_sa)

    def one_stage(s, cur, cur_gsem, cur_ssem, other, other_gsem, other_ssem):
        wait_store(s - 1, other, other_ssem)

        @pl.when(s + 1 < n_stages)
        def _():
            fire(s + 1, other, other_gsem)

        wait_gathers(s, cur, cur_gsem)
        store(s, cur, cur_ssem)

    def pair(k, carry):
        s = 2 * k + 1
        one_stage(s, rows_b, sem_b, sem_sb, rows_a, sem_a, sem_sa)
        one_stage(s + 1, rows_a, sem_a, sem_sa, rows_b, sem_b, sem_sb)
        return carry

    lax.fori_loop(0, (n_stages - 2) // 2, pair, 0, unroll=False)
    s_last = n_stages - 1
    one_stage(s_last, rows_b, sem_b, sem_sb, rows_a, sem_a, sem_sa)
    wait_store(s_last, rows_b, sem_sb)


def _sc_gather(yv, idx2d):
    n_chunks = idx2d.shape[0]
    n = n_chunks * CHUNK
    chunks_per_w = n_chunks // NW
    mesh = plsc.VectorSubcoreMesh(
        core_axis_name="c", subcore_axis_name="s", num_cores=NC, num_subcores=NS
    )
    return pl.kernel(
        _gather_body,
        out_type=jax.ShapeDtypeStruct((n, D), jnp.float32),
        mesh=mesh,
        scratch_types=[
            pltpu.VMEM((chunks_per_w, CHUNK), jnp.int32),
            pltpu.VMEM((STAGE, D), jnp.float32),
            pltpu.VMEM((STAGE, D), jnp.float32),
            pltpu.SemaphoreType.DMA,
            pltpu.SemaphoreType.DMA,
            pltpu.SemaphoreType.DMA,
            pltpu.SemaphoreType.DMA,
        ],
        compiler_params=pltpu.CompilerParams(use_tc_tiling_on_sc=False),
        name="sc_embedding_gather",
    )(yv, idx2d)


def _project_body(x_ref, w_ref, b_ref, *rest):
    o_ref = rest[-1]
    z2t = lax.dot_general(
        w_ref[...], x_ref[...], (((0,), (1,)), ((), ())),
        preferred_element_type=jnp.float32)
    y = jnp.tanh(z2t + b_ref[...])
    o_ref[0] = y[:D]
    o_ref[1] = y[D:]


def _project_chunk(g2c, Wd, b2, prev, k, pairs, B, L):
    in_specs = [
        pl.BlockSpec((B, 2 * D), lambda i: (i, 0)),
        pl.BlockSpec((2 * D, 2 * D), lambda i: (0, 0)),
        pl.BlockSpec((2 * D, 1), lambda i: (0, 0)),
    ]
    operands = [g2c, Wd, b2]
    aliases = {}
    if prev is not None:
        in_specs.append(pl.BlockSpec(memory_space=pl.ANY))
        operands.append(prev)
        aliases = {3: 0}
    return pl.pallas_call(
        _project_body,
        grid=(pairs,),
        in_specs=in_specs,
        out_specs=pl.BlockSpec((2, D, B), lambda i: (k * pairs + i, 0, 0)),
        out_shape=jax.ShapeDtypeStruct((L, D, B), jnp.float32),
        input_output_aliases=aliases,
        name="project_tanh_to_lanes",
    )(*operands)


def kernel(inputs, table, W, b):
    B, L = inputs.shape
    tableT = table.T
    ident = jnp.eye(2 * D, dtype=jnp.float32)
    y2 = _pack_table(tableT, ident)
    yv = y2.reshape(-1, D)

    idxp = inputs.T.astype(jnp.int32)
    idx_pair = jnp.transpose(
        idxp.reshape(L // 2, 2, B), (0, 2, 1)).reshape(-1)
    q = ((idx_pair >> SH_B) << SH_H) + (idx_pair & (HF - 1))
    idx_r = 2 * q + ((idx_pair >> SH_H) & 1)

    Wd = jnp.zeros((2 * D, 2 * D), jnp.float32)
    Wd = Wd.at[:D, :D].set(W).at[D:, D:].set(W)
    b2 = jnp.concatenate([b, b]).reshape(2 * D, 1)

    pairs = L // 2 // K
    rows = pairs * 2 * B
    out = None
    for k in range(K):
        idx2d = lax.slice(idx_r, (k * rows,), ((k + 1) * rows,)).reshape(
            -1, CHUNK)
        g = _sc_gather(yv, idx2d)
        g2 = g.reshape(rows // 2, 2 * D)
        out = _project_chunk(g2, Wd, b2, out, k, pairs, B, L)

    return jnp.transpose(out, (2, 0, 1))


# speedup vs baseline: 1.0911x; 1.0106x over previous
import jax
import jax.numpy as jnp
from jax import lax
from jax.experimental import pallas as pl
from jax.experimental.pallas import tpu as pltpu
from jax.experimental.pallas import tpu_sc as plsc

NC = 2
NS = 16
NW = NC * NS
D = 64
CHUNK = 128
SUB = 4
STAGE = CHUNK * SUB
K = 1


TK = 32768
HF = TK // 2
SH_B = TK.bit_length() - 1
SH_H = HF.bit_length() - 1


def _pack_body(x_ref, i_ref, y_ref):
    x = x_ref[...]
    x2 = jnp.concatenate([x[:, :HF], x[:, HF:]], axis=0)
    y_ref[...] = lax.dot_general(
        x2, i_ref[...], (((0,), (0,)), ((), ())),
        preferred_element_type=jnp.float32)


def _pack_table(tableT, ident):
    _, v = tableT.shape
    nblk = pl.cdiv(v, TK)
    return pl.pallas_call(
        _pack_body,
        grid=(nblk,),
        in_specs=[
            pl.BlockSpec((D, TK), lambda i: (0, i)),
            pl.BlockSpec((2 * D, 2 * D), lambda i: (0, 0)),
        ],
        out_specs=pl.BlockSpec((HF, 2 * D), lambda i: (i, 0)),
        out_shape=jax.ShapeDtypeStruct((nblk * HF, 2 * D), jnp.float32),
        compiler_params=pltpu.CompilerParams(fuse_transposed_lhs_in_matmul=True),
        name="table_pack_rowmajor",
    )(tableT, ident)


def _gather_body(table_hbm, idx_hbm, out_hbm,
                 idx_v, rows_a, rows_b, sem_a, sem_b, sem_sa, sem_sb):
    wid = lax.axis_index("s") * NC + lax.axis_index("c")
    chunks_per_w = idx_v.shape[0]
    n_per_w = chunks_per_w * CHUNK
    n_stages = chunks_per_w // SUB
    base = wid * n_per_w

    pltpu.sync_copy(idx_hbm.at[pl.ds(wid * chunks_per_w, chunks_per_w)], idx_v)

    def fire(stage, rows_v, sem):
        for j in range(SUB):
            pltpu.async_copy(
                table_hbm.at[idx_v.at[stage * SUB + j]],
                rows_v.at[pl.ds(j * CHUNK, CHUNK)],
                sem,
            )

    def wait_gathers(stage, rows_v, sem):
        for j in range(SUB):
            pltpu.make_async_copy(
                table_hbm.at[idx_v.at[stage * SUB + j]],
                rows_v.at[pl.ds(j * CHUNK, CHUNK)],
                sem,
            ).wait()

    def store(stage, rows_v, sem):
        pltpu.async_copy(
            rows_v, out_hbm.at[pl.ds(base + stage * STAGE, STAGE)], sem
        )

    def wait_store(stage, rows_v, sem):
        pltpu.make_async_copy(
            rows_v, out_hbm.at[pl.ds(base + stage * STAGE, STAGE)], sem
        ).wait()

    fire(0, rows_a, sem_a)
    fire(1, rows_b, sem_b)
    wait_gathers(0, rows_a, sem_a)
    store(0, rows_a, sem_sa)

    def one_stage(s, cur, cur_gsem, cur_ssem, other, other_gsem, other_ssem):
        wait_store(s - 1, other, other_ssem)

        @pl.when(s + 1 < n_stages)
        def _():
            fire(s + 1, other, other_gsem)

        wait_gathers(s, cur, cur_gsem)
        store(s, cur, cur_ssem)

    def pair(k, carry):
        s = 2 * k + 1
        one_stage(s, rows_b, sem_b, sem_sb, rows_a, sem_a, sem_sa)
        one_stage(s + 1, rows_a, sem_a, sem_sa, rows_b, sem_b, sem_sb)
        return carry

    lax.fori_loop(0, (n_stages - 2) // 2, pair, 0, unroll=False)
    s_last = n_stages - 1
    one_stage(s_last, rows_b, sem_b, sem_sb, rows_a, sem_a, sem_sa)
    wait_store(s_last, rows_b, sem_sb)


def _sc_gather(yv, idx2d):
    n_chunks = idx2d.shape[0]
    n = n_chunks * CHUNK
    chunks_per_w = n_chunks // NW
    mesh = plsc.VectorSubcoreMesh(
        core_axis_name="c", subcore_axis_name="s", num_cores=NC, num_subcores=NS
    )
    return pl.kernel(
        _gather_body,
        out_type=jax.ShapeDtypeStruct((n, D), jnp.float32),
        mesh=mesh,
        scratch_types=[
            pltpu.VMEM((chunks_per_w, CHUNK), jnp.int32),
            pltpu.VMEM((STAGE, D), jnp.float32),
            pltpu.VMEM((STAGE, D), jnp.float32),
            pltpu.SemaphoreType.DMA,
            pltpu.SemaphoreType.DMA,
            pltpu.SemaphoreType.DMA,
            pltpu.SemaphoreType.DMA,
        ],
        compiler_params=pltpu.CompilerParams(use_tc_tiling_on_sc=False),
        name="sc_embedding_gather",
    )(yv, idx2d)


def _project_body(x_ref, w_ref, b_ref, *rest):
    o_ref = rest[-1]
    z2t = lax.dot_general(
        w_ref[...], x_ref[...], (((0,), (1,)), ((), ())),
        preferred_element_type=jnp.float32)
    y = jnp.tanh(z2t + b_ref[...])
    o_ref[0] = y[:D]
    o_ref[1] = y[D:]


def _project_chunk(g2c, Wd, b2, prev, k, pairs, B, L):
    in_specs = [
        pl.BlockSpec((B, 2 * D), lambda i: (i, 0)),
        pl.BlockSpec((2 * D, 2 * D), lambda i: (0, 0)),
        pl.BlockSpec((2 * D, 1), lambda i: (0, 0)),
    ]
    operands = [g2c, Wd, b2]
    aliases = {}
    if prev is not None:
        in_specs.append(pl.BlockSpec(memory_space=pl.ANY))
        operands.append(prev)
        aliases = {3: 0}
    return pl.pallas_call(
        _project_body,
        grid=(pairs,),
        in_specs=in_specs,
        out_specs=pl.BlockSpec((2, D, B), lambda i: (k * pairs + i, 0, 0)),
        out_shape=jax.ShapeDtypeStruct((L, D, B), jnp.float32),
        input_output_aliases=aliases,
        name="project_tanh_to_lanes",
    )(*operands)


def kernel(inputs, table, W, b):
    B, L = inputs.shape
    tableT = table.T
    ident = jnp.eye(2 * D, dtype=jnp.float32)
    y2 = _pack_table(tableT, ident)
    yv = y2.reshape(-1, D)

    idxp = inputs.T.astype(jnp.int32)
    idx_pair = jnp.transpose(
        idxp.reshape(L // 2, 2, B), (0, 2, 1)).reshape(-1)
    q = ((idx_pair >> SH_B) << SH_H) + (idx_pair & (HF - 1))
    idx_r = 2 * q + ((idx_pair >> SH_H) & 1)

    Wd = jnp.zeros((2 * D, 2 * D), jnp.float32)
    Wd = Wd.at[:D, :D].set(W).at[D:, D:].set(W)
    b2 = jnp.concatenate([b, b]).reshape(2 * D, 1)

    pairs = L // 2 // K
    rows = pairs * 2 * B
    out = None
    for k in range(K):
        idx2d = lax.slice(idx_r, (k * rows,), ((k + 1) * rows,)).reshape(
            -1, CHUNK)
        g = _sc_gather(yv, idx2d)
        g2 = g.reshape(rows // 2, 2 * D)
        out = _project_chunk(g2, Wd, b2, out, k, pairs, B, L)

    return jnp.transpose(out, (2, 0, 1))


# speedup vs baseline: 1.1220x; 1.0284x over previous
import jax
import jax.numpy as jnp
from jax import lax
from jax.experimental import pallas as pl
from jax.experimental.pallas import tpu as pltpu
from jax.experimental.pallas import tpu_sc as plsc

NC = 2
NS = 16
NW = NC * NS
D = 64
CHUNK = 128
SUB = 4
STAGE = CHUNK * SUB
K = 1


TK = 32768
HF = TK // 2
SH_B = TK.bit_length() - 1
SH_H = HF.bit_length() - 1


def _pack_body(x_ref, i_ref, y_ref):
    x = x_ref[...]
    x2 = jnp.concatenate([x[:, :HF], x[:, HF:]], axis=0)
    y_ref[...] = lax.dot_general(
        x2, i_ref[...], (((0,), (0,)), ((), ())),
        preferred_element_type=jnp.float32)


def _pack_table(tableT, ident):
    _, v = tableT.shape
    nblk = pl.cdiv(v, TK)
    return pl.pallas_call(
        _pack_body,
        grid=(nblk,),
        in_specs=[
            pl.BlockSpec((D, TK), lambda i: (0, i)),
            pl.BlockSpec((2 * D, 2 * D), lambda i: (0, 0)),
        ],
        out_specs=pl.BlockSpec((HF, 2 * D), lambda i: (i, 0)),
        out_shape=jax.ShapeDtypeStruct((nblk * HF, 2 * D), jnp.float32),
        compiler_params=pltpu.CompilerParams(fuse_transposed_lhs_in_matmul=True),
        name="table_pack_rowmajor",
    )(tableT, ident)


def _gather_body(table_hbm, idx_hbm, out_hbm,
                 idx_v, rows_a, rows_b, sem_a, sem_b, sem_sa, sem_sb):
    wid = lax.axis_index("s") * NC + lax.axis_index("c")
    chunks_per_w = idx_v.shape[0]
    n_per_w = chunks_per_w * CHUNK
    n_stages = chunks_per_w // SUB
    base = wid * n_per_w

    pltpu.sync_copy(idx_hbm.at[pl.ds(wid * chunks_per_w, chunks_per_w)], idx_v)

    def fire(stage, rows_v, sem):
        for j in range(SUB):
            pltpu.async_copy(
                table_hbm.at[idx_v.at[stage * SUB + j]],
                rows_v.at[pl.ds(j * CHUNK, CHUNK)],
                sem,
            )

    def wait_gathers(stage, rows_v, sem):
        for j in range(SUB):
            pltpu.make_async_copy(
                table_hbm.at[idx_v.at[stage * SUB + j]],
                rows_v.at[pl.ds(j * CHUNK, CHUNK)],
                sem,
            ).wait()

    def store(stage, rows_v, sem):
        pltpu.async_copy(
            rows_v, out_hbm.at[pl.ds(base + stage * STAGE, STAGE)], sem
        )

    def wait_store(stage, rows_v, sem):
        pltpu.make_async_copy(
            rows_v, out_hbm.at[pl.ds(base + stage * STAGE, STAGE)], sem
        ).wait()

    fire(0, rows_a, sem_a)
    fire(1, rows_b, sem_b)
    wait_gathers(0, rows_a, sem_a)
    store(0, rows_a, sem_sa)

    def one_stage(s, cur, cur_gsem, cur_ssem, other, other_gsem, other_ssem):
        wait_store(s - 1, other, other_ssem)

        @pl.when(s + 1 < n_stages)
        def _():
            fire(s + 1, other, other_gsem)

        wait_gathers(s, cur, cur_gsem)
        store(s, cur, cur_ssem)

    def pair(k, carry):
        s = 2 * k + 1
        one_stage(s, rows_b, sem_b, sem_sb, rows_a, sem_a, sem_sa)
        one_stage(s + 1, rows_a, sem_a, sem_sa, rows_b, sem_b, sem_sb)
        return carry

    lax.fori_loop(0, (n_stages - 2) // 2, pair, 0, unroll=False)
    s_last = n_stages - 1
    one_stage(s_last, rows_b, sem_b, sem_sb, rows_a, sem_a, sem_sa)
    wait_store(s_last, rows_b, sem_sb)


def _sc_gather(yv, idx2d):
    n_chunks = idx2d.shape[0]
    n = n_chunks * CHUNK
    chunks_per_w = n_chunks // NW
    mesh = plsc.VectorSubcoreMesh(
        core_axis_name="c", subcore_axis_name="s", num_cores=NC, num_subcores=NS
    )
    return pl.kernel(
        _gather_body,
        out_type=jax.ShapeDtypeStruct((n, D), jnp.float32),
        mesh=mesh,
        scratch_types=[
            pltpu.VMEM((chunks_per_w, CHUNK), jnp.int32),
            pltpu.VMEM((STAGE, D), jnp.float32),
            pltpu.VMEM((STAGE, D), jnp.float32),
            pltpu.SemaphoreType.DMA,
            pltpu.SemaphoreType.DMA,
            pltpu.SemaphoreType.DMA,
            pltpu.SemaphoreType.DMA,
        ],
        compiler_params=pltpu.CompilerParams(use_tc_tiling_on_sc=False),
        name="sc_embedding_gather",
    )(yv, idx2d)


def _project_body(x_ref, w_ref, b_ref, *rest):
    o_ref = rest[-1]
    B = x_ref.shape[0] // 2
    z2t = lax.dot_general(
        w_ref[...], x_ref[...], (((0,), (1,)), ((), ())),
        preferred_element_type=jnp.float32)
    y = jnp.tanh(z2t + b_ref[...])
    o_ref[0] = y[:D, :B]
    o_ref[1] = y[D:, :B]
    o_ref[2] = y[:D, B:]
    o_ref[3] = y[D:, B:]


def _project_chunk(g2c, Wd, b2, prev, k, pairs, B, L):
    in_specs = [
        pl.BlockSpec((2 * B, 2 * D), lambda i: (i, 0)),
        pl.BlockSpec((2 * D, 2 * D), lambda i: (0, 0)),
        pl.BlockSpec((2 * D, 1), lambda i: (0, 0)),
    ]
    operands = [g2c, Wd, b2]
    aliases = {}
    if prev is not None:
        in_specs.append(pl.BlockSpec(memory_space=pl.ANY))
        operands.append(prev)
        aliases = {3: 0}
    return pl.pallas_call(
        _project_body,
        grid=(pairs // 2,),
        in_specs=in_specs,
        out_specs=pl.BlockSpec(
            (4, D, B), lambda i: (k * (pairs // 2) + i, 0, 0)),
        out_shape=jax.ShapeDtypeStruct((L, D, B), jnp.float32),
        input_output_aliases=aliases,
        name="project_tanh_to_lanes",
    )(*operands)


def kernel(inputs, table, W, b):
    B, L = inputs.shape
    tableT = table.T
    ident = jnp.eye(2 * D, dtype=jnp.float32)
    y2 = _pack_table(tableT, ident)
    yv = y2.reshape(-1, D)

    idxp = inputs.T.astype(jnp.int32)
    idx_pair = jnp.transpose(
        idxp.reshape(L // 2, 2, B), (0, 2, 1)).reshape(-1)
    q = ((idx_pair >> SH_B) << SH_H) + (idx_pair & (HF - 1))
    idx_r = 2 * q + ((idx_pair >> SH_H) & 1)

    Wd = jnp.zeros((2 * D, 2 * D), jnp.float32)
    Wd = Wd.at[:D, :D].set(W).at[D:, D:].set(W)
    b2 = jnp.concatenate([b, b]).reshape(2 * D, 1)

    pairs = L // 2 // K
    rows = pairs * 2 * B
    out = None
    for k in range(K):
        idx2d = lax.slice(idx_r, (k * rows,), ((k + 1) * rows,)).reshape(
            -1, CHUNK)
        g = _sc_gather(yv, idx2d)
        g2 = g.reshape(rows // 2, 2 * D)
        out = _project_chunk(g2, Wd, b2, out, k, pairs, B, L)

    return jnp.transpose(out, (2, 0, 1))


# speedup vs baseline: 1.1271x; 1.0045x over previous
import jax
import jax.numpy as jnp
from jax import lax
from jax.experimental import pallas as pl
from jax.experimental.pallas import tpu as pltpu
from jax.experimental.pallas import tpu_sc as plsc

NC = 2
NS = 16
NW = NC * NS
D = 64
CHUNK = 128
SUB = 4
STAGE = CHUNK * SUB
K = 1


TK = 32768
HF = TK // 2
SH_B = TK.bit_length() - 1
SH_H = HF.bit_length() - 1


def _pack_body(x_ref, i_ref, y_ref):
    x = x_ref[...]
    x2 = jnp.concatenate([x[:, :HF], x[:, HF:]], axis=0)
    y_ref[...] = lax.dot_general(
        x2, i_ref[...], (((0,), (0,)), ((), ())),
        preferred_element_type=jnp.float32)


def _pack_table(tableT, ident):
    _, v = tableT.shape
    nblk = pl.cdiv(v, TK)
    return pl.pallas_call(
        _pack_body,
        grid=(nblk,),
        in_specs=[
            pl.BlockSpec((D, TK), lambda i: (0, i)),
            pl.BlockSpec((2 * D, 2 * D), lambda i: (0, 0)),
        ],
        out_specs=pl.BlockSpec((HF, 2 * D), lambda i: (i, 0)),
        out_shape=jax.ShapeDtypeStruct((nblk * HF, 2 * D), jnp.float32),
        compiler_params=pltpu.CompilerParams(fuse_transposed_lhs_in_matmul=True),
        name="table_pack_rowmajor",
    )(tableT, ident)


def _gather_body(table_hbm, idx_hbm, out_hbm,
                 idx_v, rows_a, rows_b, sem_a, sem_b, sem_sa, sem_sb):
    wid = lax.axis_index("s") * NC + lax.axis_index("c")
    chunks_per_w = idx_v.shape[0]
    n_per_w = chunks_per_w * CHUNK
    n_stages = chunks_per_w // SUB
    base = wid * n_per_w

    pltpu.sync_copy(idx_hbm.at[pl.ds(wid * chunks_per_w, chunks_per_w)], idx_v)

    def fire(stage, rows_v, sem):
        for j in range(SUB):
            pltpu.async_copy(
                table_hbm.at[idx_v.at[stage * SUB + j]],
                rows_v.at[pl.ds(j * CHUNK, CHUNK)],
                sem,
            )

    def wait_gathers(stage, rows_v, sem):
        for j in range(SUB):
            pltpu.make_async_copy(
                table_hbm.at[idx_v.at[stage * SUB + j]],
                rows_v.at[pl.ds(j * CHUNK, CHUNK)],
                sem,
            ).wait()

    def store(stage, rows_v, sem):
        pltpu.async_copy(
            rows_v, out_hbm.at[pl.ds(base + stage * STAGE, STAGE)], sem
        )

    def wait_store(stage, rows_v, sem):
        pltpu.make_async_copy(
            rows_v, out_hbm.at[pl.ds(base + stage * STAGE, STAGE)], sem
        ).wait()

    fire(0, rows_a, sem_a)
    fire(1, rows_b, sem_b)
    wait_gathers(0, rows_a, sem_a)
    store(0, rows_a, sem_sa)

    def one_stage(s, cur, cur_gsem, cur_ssem, other, other_gsem, other_ssem):
        wait_store(s - 1, other, other_ssem)

        @pl.when(s + 1 < n_stages)
        def _():
            fire(s + 1, other, other_gsem)

        wait_gathers(s, cur, cur_gsem)
        store(s, cur, cur_ssem)

    def pair(k, carry):
        s = 2 * k + 1
        one_stage(s, rows_b, sem_b, sem_sb, rows_a, sem_a, sem_sa)
        one_stage(s + 1, rows_a, sem_a, sem_sa, rows_b, sem_b, sem_sb)
        return carry

    lax.fori_loop(0, (n_stages - 2) // 2, pair, 0, unroll=False)
    s_last = n_stages - 1
    one_stage(s_last, rows_b, sem_b, sem_sb, rows_a, sem_a, sem_sa)
    wait_store(s_last, rows_b, sem_sb)


def _sc_gather(yv, idx2d):
    n_chunks = idx2d.shape[0]
    n = n_chunks * CHUNK
    chunks_per_w = n_chunks // NW
    mesh = plsc.VectorSubcoreMesh(
        core_axis_name="c", subcore_axis_name="s", num_cores=NC, num_subcores=NS
    )
    return pl.kernel(
        _gather_body,
        out_type=jax.ShapeDtypeStruct((n, D), jnp.float32),
        mesh=mesh,
        scratch_types=[
            pltpu.VMEM((chunks_per_w, CHUNK), jnp.int32),
            pltpu.VMEM((STAGE, D), jnp.float32),
            pltpu.VMEM((STAGE, D), jnp.float32),
            pltpu.SemaphoreType.DMA,
            pltpu.SemaphoreType.DMA,
            pltpu.SemaphoreType.DMA,
            pltpu.SemaphoreType.DMA,
        ],
        compiler_params=pltpu.CompilerParams(use_tc_tiling_on_sc=False),
        name="sc_embedding_gather",
    )(yv, idx2d)


PP = 4


def _project_body(x_ref, w_ref, b_ref, *rest):
    o_ref = rest[-1]
    B = x_ref.shape[0] // PP
    z2t = lax.dot_general(
        w_ref[...], x_ref[...], (((0,), (1,)), ((), ())),
        preferred_element_type=jnp.float32)
    y = jnp.tanh(z2t + b_ref[...])
    for p in range(PP):
        o_ref[2 * p] = y[:D, p * B:(p + 1) * B]
        o_ref[2 * p + 1] = y[D:, p * B:(p + 1) * B]


def _project_chunk(g2c, Wd, b2, prev, k, pairs, B, L):
    in_specs = [
        pl.BlockSpec((PP * B, 2 * D), lambda i: (i, 0)),
        pl.BlockSpec((2 * D, 2 * D), lambda i: (0, 0)),
        pl.BlockSpec((2 * D, 1), lambda i: (0, 0)),
    ]
    operands = [g2c, Wd, b2]
    aliases = {}
    if prev is not None:
        in_specs.append(pl.BlockSpec(memory_space=pl.ANY))
        operands.append(prev)
        aliases = {3: 0}
    return pl.pallas_call(
        _project_body,
        grid=(pairs // PP,),
        in_specs=in_specs,
        out_specs=pl.BlockSpec(
            (2 * PP, D, B), lambda i: (k * (pairs // PP) + i, 0, 0)),
        out_shape=jax.ShapeDtypeStruct((L, D, B), jnp.float32),
        input_output_aliases=aliases,
        name="project_tanh_to_lanes",
    )(*operands)


def kernel(inputs, table, W, b):
    B, L = inputs.shape
    tableT = table.T
    ident = jnp.eye(2 * D, dtype=jnp.float32)
    y2 = _pack_table(tableT, ident)
    yv = y2.reshape(-1, D)

    idxp = inputs.T.astype(jnp.int32)
    idx_pair = jnp.transpose(
        idxp.reshape(L // 2, 2, B), (0, 2, 1)).reshape(-1)
    q = ((idx_pair >> SH_B) << SH_H) + (idx_pair & (HF - 1))
    idx_r = 2 * q + ((idx_pair >> SH_H) & 1)

    Wd = jnp.zeros((2 * D, 2 * D), jnp.float32)
    Wd = Wd.at[:D, :D].set(W).at[D:, D:].set(W)
    b2 = jnp.concatenate([b, b]).reshape(2 * D, 1)

    pairs = L // 2 // K
    rows = pairs * 2 * B
    out = None
    for k in range(K):
        idx2d = lax.slice(idx_r, (k * rows,), ((k + 1) * rows,)).reshape(
            -1, CHUNK)
        g = _sc_gather(yv, idx2d)
        g2 = g.reshape(rows // 2, 2 * D)
        out = _project_chunk(g2, Wd, b2, out, k, pairs, B, L)

    return jnp.transpose(out, (2, 0, 1))
